# speedup vs baseline: 1.3262x; 1.3262x over previous
import functools

import jax
import jax.numpy as jnp
from jax import lax
from jax.experimental import pallas as pl
from jax.experimental.pallas import tpu as pltpu
from jax.experimental.pallas import tpu_sc as plsc

_VOCAB = 1000
_D1 = 32
_D2 = 16
_B = 16384
_R1 = _D1 * _D1
_R2 = _D2 * _D2

_NC = 2
_NS = 16
_NW = _NC * _NS
_BPW = _B // _NW
_C = 64
_NCHUNK = _BPW // _C


def _sc_gather(x, t1, t2):
  mesh = plsc.VectorSubcoreMesh(core_axis_name="c", subcore_axis_name="s")

  @functools.partial(
      pl.kernel,
      out_type=[
          jax.ShapeDtypeStruct((_B, _R1), jnp.float32),
          jax.ShapeDtypeStruct((_B, _R2), jnp.float32),
      ],
      mesh=mesh,
      scratch_types=[
          pltpu.VMEM((_C,), jnp.int32),
          pltpu.VMEM((_C, _R1), jnp.float32),
          pltpu.VMEM((_C, _R2), jnp.float32),
          pltpu.SemaphoreType.DMA,
          pltpu.SemaphoreType.DMA,
      ],
  )
  def body(x_hbm, t1_hbm, t2_hbm, o1_hbm, o2_hbm, idx_v, buf1, buf2, s1, s2):
    wid = lax.axis_index("s") * _NC + lax.axis_index("c")
    base = wid * _BPW
    for c in range(_NCHUNK):
      cb = base + c * _C
      pltpu.sync_copy(x_hbm.at[pl.ds(cb, _C)], idx_v)
      cp1 = pltpu.async_copy(t1_hbm.at[idx_v], buf1, s1)
      cp2 = pltpu.async_copy(t2_hbm.at[idx_v], buf2, s2)
      cp1.wait()
      pltpu.sync_copy(buf1, o1_hbm.at[pl.ds(cb, _C)])
      cp2.wait()
      pltpu.sync_copy(buf2, o2_hbm.at[pl.ds(cb, _C)])

  return body(x, t1, t2)


@jax.jit
def kernel(x, T1, T2):
  t1 = T1.reshape(_VOCAB, _R1)
  t2 = T2.reshape(_VOCAB, _R2)
  o1, o2 = _sc_gather(x.astype(jnp.int32), t1, t2)
  return (o1.reshape(_B * _D1, _D1), o2.reshape(_B * _D2, _D2))


# speedup vs baseline: 1.3475x; 1.0160x over previous
import functools

import jax
import jax.numpy as jnp
from jax import lax
from jax.experimental import pallas as pl
from jax.experimental.pallas import tpu as pltpu
from jax.experimental.pallas import tpu_sc as plsc

_VOCAB = 1000
_D1 = 32
_D2 = 16
_B = 16384
_R1 = _D1 * _D1
_R2 = _D2 * _D2

_NC = 2
_NS = 16
_NW = _NC * _NS
_BPW = _B // _NW
_C = 32
_NCHUNK = _BPW // _C


def _sc_gather(x, t1, t2):
  mesh = plsc.VectorSubcoreMesh(core_axis_name="c", subcore_axis_name="s")

  @functools.partial(
      pl.kernel,
      out_type=[
          jax.ShapeDtypeStruct((_B, _R1), jnp.float32),
          jax.ShapeDtypeStruct((_B, _R2), jnp.float32),
      ],
      mesh=mesh,
      scratch_types=[
          pltpu.VMEM((_BPW,), jnp.int32),
          pltpu.VMEM((2, _C, _R1), jnp.float32),
          pltpu.VMEM((2, _C, _R2), jnp.float32),
          pltpu.SemaphoreType.DMA,
          pltpu.SemaphoreType.DMA,
      ],
  )
  def body(x_hbm, t1_hbm, t2_hbm, o1_hbm, o2_hbm, idx_v, buf1, buf2, g0, g1):
    wid = lax.axis_index("s") * _NC + lax.axis_index("c")
    base = wid * _BPW
    gsem = (g0, g1)
    pltpu.sync_copy(x_hbm.at[pl.ds(base, _BPW)], idx_v)

    def fire(c):
      p = c & 1
      i = idx_v.at[pl.ds(c * _C, _C)]
      return (pltpu.async_copy(t1_hbm.at[i], buf1.at[p], gsem[p]),
              pltpu.async_copy(t2_hbm.at[i], buf2.at[p], gsem[p]))

    pending = fire(0)
    for c in range(_NCHUNK):
      p = c & 1
      cb = base + c * _C
      nxt = fire(c + 1) if c + 1 < _NCHUNK else None
      pending[0].wait()
      pending[1].wait()
      pltpu.sync_copy(buf1.at[p], o1_hbm.at[pl.ds(cb, _C)])
      pltpu.sync_copy(buf2.at[p], o2_hbm.at[pl.ds(cb, _C)])
      pending = nxt

  return body(x, t1, t2)


@jax.jit
def kernel(x, T1, T2):
  t1 = T1.reshape(_VOCAB, _R1)
  t2 = T2.reshape(_VOCAB, _R2)
  o1, o2 = _sc_gather(x.astype(jnp.int32), t1, t2)
  return (o1.reshape(_B * _D1, _D1), o2.reshape(_B * _D2, _D2))
